# baseline (speedup 1.0000x reference)
import jax
import jax.numpy as jnp
from jax.experimental import pallas as pl

_ROWS = 1_000_000
_DIM = 64
_WROWS = 500_000
_WDIM = 128
_BLOCK_ROWS = 4_000


def _copy_block(in_ref, out_ref):
    out_ref[...] = in_ref[...]


def kernel(path_emb):
    wide = jnp.reshape(path_emb, (_WROWS, _WDIM))
    out = pl.pallas_call(
        _copy_block,
        grid=(_WROWS // _BLOCK_ROWS,),
        in_specs=[pl.BlockSpec((_BLOCK_ROWS, _WDIM), lambda i: (i, 0))],
        out_specs=pl.BlockSpec((_BLOCK_ROWS, _WDIM), lambda i: (i, 0)),
        out_shape=jax.ShapeDtypeStruct((_WROWS, _WDIM), jnp.float32),
    )(wide)
    return jnp.reshape(out, (_ROWS, _DIM))
